# baseline (speedup 1.0000x reference)
import jax
import jax.numpy as jnp
from jax.experimental import pallas as pl
from jax.experimental.pallas import tpu as pltpu

B = 256
IN_DIM = 512
OUT_DIM = 10000
WBLK = 1920
NW = -(-OUT_DIM // WBLK)
BJ = 400
NJ = OUT_DIM // BJ


def _gcn_kernel(x_ref, w_ref, adj_ref, bias_ref, out_ref, sT_ref):
    j = pl.program_id(0)

    @pl.when(j < NW)
    def _():
        chunk = jnp.dot(
            x_ref[...].astype(jnp.bfloat16),
            w_ref[...].astype(jnp.bfloat16),
            preferred_element_type=jnp.float32,
        )
        sT_ref[pl.ds(j * WBLK, WBLK), :] = chunk.T.astype(jnp.bfloat16)

    @pl.when(j >= NW)
    def _():
        out_ref[...] = (
            jnp.dot(
                adj_ref[...].astype(jnp.bfloat16),
                sT_ref[pl.ds(0, OUT_DIM), :],
                preferred_element_type=jnp.float32,
            )
            + bias_ref[...]
        )


def kernel(input, adj, weight, bias):
    outT = pl.pallas_call(
        _gcn_kernel,
        grid=(NW + NJ,),
        in_specs=[
            pl.BlockSpec((B, IN_DIM), lambda j: (0, 0)),
            pl.BlockSpec((IN_DIM, WBLK), lambda j: (0, jnp.minimum(j, NW - 1))),
            pl.BlockSpec((BJ, OUT_DIM), lambda j: (jnp.maximum(j - NW, 0), 0)),
            pl.BlockSpec((BJ, 1), lambda j: (jnp.maximum(j - NW, 0), 0)),
        ],
        out_specs=pl.BlockSpec((BJ, B), lambda j: (jnp.maximum(j - NW, 0), 0)),
        out_shape=jax.ShapeDtypeStruct((OUT_DIM, B), jnp.float32),
        scratch_shapes=[pltpu.VMEM((NW * WBLK, B), jnp.bfloat16)],
    )(input, weight, adj, bias.reshape(OUT_DIM, 1))
    return outT.T
